# Initial kernel scaffold; baseline (speedup 1.0000x reference)
import jax
import jax.numpy as jnp
from jax.experimental import pallas as pl


def kernel(x, edge_index, edge_type, edge_norm, dim, weight_relation, root):
    raise NotImplementedError("write your pallas kernel here")


# speedup vs baseline: 10.8703x; 10.8703x over previous
import functools

import jax
import jax.numpy as jnp
from jax import lax
from jax.experimental import pallas as pl
from jax.experimental.pallas import tpu as pltpu
from jax.experimental.pallas import tpu_sc as plsc

N = 10000
E = 320000
D = 128
R = 8
NC = 2
NS = 16
NW = NC * NS
EW = E // NW
C = 80
NCHUNK = EW // C
ZSTRIPE = 624
ZTAIL = N - NS * ZSTRIPE


def _h_body(x_ref, w_ref, h_ref):
    h_ref[0] = jnp.dot(x_ref[...], w_ref[0], preferred_element_type=jnp.float32)


def _compute_h(x, weight_relation):
    bn = 2000
    return pl.pallas_call(
        _h_body,
        grid=(R, N // bn),
        in_specs=[
            pl.BlockSpec((bn, D), lambda r, i: (i, 0)),
            pl.BlockSpec((1, D, D), lambda r, i: (r, 0, 0)),
        ],
        out_specs=pl.BlockSpec((1, bn, D), lambda r, i: (r, i, 0)),
        out_shape=jax.ShapeDtypeStruct((R, N, D), jnp.float32),
    )(x, weight_relation)


def _sc_body(h_hbm, et_hbm, src_hbm, dst_hbm, nrm_hbm, zeros_hbm, part_hbm,
             et_v, src_v, idx_v, dst_v, nrm_v, rows_v, accum_sh, sem):
    c = lax.axis_index("c")
    s = lax.axis_index("s")
    wid = c * NS + s

    zstart = pl.multiple_of(s * ZSTRIPE, 8)
    pltpu.sync_copy(zeros_hbm.at[pl.ds(zstart, ZSTRIPE)],
                    accum_sh.at[pl.ds(zstart, ZSTRIPE)])

    @pl.when(s == NS - 1)
    def _():
        pltpu.sync_copy(zeros_hbm.at[pl.ds(NS * ZSTRIPE, ZTAIL)],
                        accum_sh.at[pl.ds(NS * ZSTRIPE, ZTAIL)])

    plsc.subcore_barrier()

    base_w = wid * EW

    def chunk(ci, carry):
        base = base_w + ci * C
        pltpu.sync_copy(et_hbm.at[pl.ds(base, C)], et_v)
        pltpu.sync_copy(src_hbm.at[pl.ds(base, C)], src_v)
        pltpu.sync_copy(dst_hbm.at[pl.ds(base, C)], dst_v)
        pltpu.sync_copy(nrm_hbm.at[pl.ds(base, C)], nrm_v)
        for g in range(C // 16):
            sl = pl.ds(g * 16, 16)
            idx_v[sl] = et_v[sl] * N + src_v[sl]
        pltpu.async_copy(h_hbm.at[idx_v], rows_v, sem).wait()

        def scale(g, carry2):
            nrm16 = nrm_v[pl.ds(g * 16, 16)]
            for l in range(16):
                e = g * 16 + l
                nv = nrm16[l]
                for j in range(D // 16):
                    sl = pl.ds(j * 16, 16)
                    rows_v[e, sl] = rows_v[e, sl] * nv
            return carry2

        lax.fori_loop(0, C // 16, scale, 0)
        pltpu.sync_copy(rows_v, accum_sh.at[dst_v], add=True)
        return carry

    lax.fori_loop(0, NCHUNK, chunk, 0)
    plsc.subcore_barrier()
    dstart = pl.multiple_of(c * N + s * ZSTRIPE, 8)
    pltpu.sync_copy(accum_sh.at[pl.ds(zstart, ZSTRIPE)],
                    part_hbm.at[pl.ds(dstart, ZSTRIPE)])

    @pl.when(s == NS - 1)
    def _():
        tstart = pl.multiple_of(c * N + NS * ZSTRIPE, 8)
        pltpu.sync_copy(accum_sh.at[pl.ds(NS * ZSTRIPE, ZTAIL)],
                        part_hbm.at[pl.ds(tstart, ZTAIL)])


_sc_scatter = pl.kernel(
    _sc_body,
    out_type=jax.ShapeDtypeStruct((NC * N, D), jnp.float32),
    mesh=plsc.VectorSubcoreMesh(core_axis_name="c", subcore_axis_name="s"),
    scratch_types=[
        pltpu.VMEM((C,), jnp.int32),
        pltpu.VMEM((C,), jnp.int32),
        pltpu.VMEM((C,), jnp.int32),
        pltpu.VMEM((C,), jnp.int32),
        pltpu.VMEM((C,), jnp.float32),
        pltpu.VMEM((C, D), jnp.float32),
        pltpu.VMEM_SHARED((N, D), jnp.float32),
        pltpu.SemaphoreType.DMA,
    ],
)


def _combine_body(p0_ref, p1_ref, x_ref, root_ref, out_ref):
    out_ref[...] = (p0_ref[...] + p1_ref[...]
                    + jnp.dot(x_ref[...], root_ref[...],
                              preferred_element_type=jnp.float32))


def _combine(part, x, root):
    bn = 2000
    nb = N // bn
    return pl.pallas_call(
        _combine_body,
        grid=(nb,),
        in_specs=[
            pl.BlockSpec((bn, D), lambda i: (i, 0)),
            pl.BlockSpec((bn, D), lambda i: (nb + i, 0)),
            pl.BlockSpec((bn, D), lambda i: (i, 0)),
            pl.BlockSpec((D, D), lambda i: (0, 0)),
        ],
        out_specs=pl.BlockSpec((bn, D), lambda i: (i, 0)),
        out_shape=jax.ShapeDtypeStruct((N, D), jnp.float32),
    )(part, part, x, root)


def kernel(x, edge_index, edge_type, edge_norm, dim, weight_relation, root):
    h = _compute_h(x, weight_relation)
    h_flat = h.reshape(R * N, D)
    src = edge_index[1].astype(jnp.int32)
    dst = edge_index[0].astype(jnp.int32)
    et = edge_type.astype(jnp.int32)
    zeros = jnp.zeros((N, D), jnp.float32)
    part = _sc_scatter(h_flat, et, src, dst, edge_norm, zeros)
    return _combine(part, x, root)


# speedup vs baseline: 27.6280x; 2.5416x over previous
import jax
import jax.numpy as jnp
from jax import lax
from jax.experimental import pallas as pl
from jax.experimental.pallas import tpu as pltpu
from jax.experimental.pallas import tpu_sc as plsc

N = 10000
E = 320000
D = 128
R = 8
NC = 2
NS = 16
NW = NC * NS
EW = E // NW
C = 80
NCHUNK = EW // C
NBUF = 3
ZSTRIPE = 624
ZTAIL = N - NS * ZSTRIPE


def _h_body(x_ref, w_ref, h_ref):
    h_ref[0] = jnp.dot(x_ref[...], w_ref[0], preferred_element_type=jnp.float32)


def _compute_h(x, weight_relation):
    bn = 2000
    return pl.pallas_call(
        _h_body,
        grid=(R, N // bn),
        in_specs=[
            pl.BlockSpec((bn, D), lambda r, i: (i, 0)),
            pl.BlockSpec((1, D, D), lambda r, i: (r, 0, 0)),
        ],
        out_specs=pl.BlockSpec((1, bn, D), lambda r, i: (r, i, 0)),
        out_shape=jax.ShapeDtypeStruct((R, N, D), jnp.float32),
    )(x, weight_relation)


def _sc_body(h_hbm, idx_hbm, dst_hbm, nrm_hbm, zeros_hbm, part_hbm,
             idx0, idx1, idx2, dst0, dst1, dst2, nrm0, nrm1, nrm2,
             rows0, rows1, rows2, accum_sh,
             gsem0, gsem1, gsem2, ssem0, ssem1, ssem2, esem0, esem1, esem2):
    c = lax.axis_index("c")
    s = lax.axis_index("s")
    wid = c * NS + s
    idxs = (idx0, idx1, idx2)
    dsts = (dst0, dst1, dst2)
    nrms = (nrm0, nrm1, nrm2)
    rows = (rows0, rows1, rows2)
    gsems = (gsem0, gsem1, gsem2)
    ssems = (ssem0, ssem1, ssem2)
    esems = (esem0, esem1, esem2)

    zstart = pl.multiple_of(s * ZSTRIPE, 8)
    pltpu.sync_copy(zeros_hbm.at[pl.ds(zstart, ZSTRIPE)],
                    accum_sh.at[pl.ds(zstart, ZSTRIPE)])

    @pl.when(s == NS - 1)
    def _():
        pltpu.sync_copy(zeros_hbm.at[pl.ds(NS * ZSTRIPE, ZTAIL)],
                        accum_sh.at[pl.ds(NS * ZSTRIPE, ZTAIL)])

    def issue_edges(ci, u):
        pltpu.async_copy(idx_hbm.at[wid, ci], idxs[u], esems[u])
        pltpu.async_copy(dst_hbm.at[wid, ci], dsts[u], esems[u])
        pltpu.async_copy(nrm_hbm.at[wid, ci], nrms[u], esems[u])

    def drain_edges(u):
        pltpu.make_async_copy(idx_hbm.at[0, 0], idxs[u], esems[u]).wait()
        pltpu.make_async_copy(dst_hbm.at[0, 0], dsts[u], esems[u]).wait()
        pltpu.make_async_copy(nrm_hbm.at[0, 0], nrms[u], esems[u]).wait()

    def drain_rows(sem, u):
        pltpu.make_async_copy(h_hbm.at[pl.ds(0, C)], rows[u], sem).wait()

    def do_scale(rows_b, nrm_b):
        def scale(g, cr):
            nrm16 = nrm_b[pl.ds(g * 16, 16)]
            for l in range(16):
                e = g * 16 + l
                nv = nrm16[l]
                for j in range(D // 16):
                    sl = pl.ds(j * 16, 16)
                    rows_b[e, sl] = rows_b[e, sl] * nv
            return cr

        lax.fori_loop(0, C // 16, scale, 0)

    for t in range(NBUF):
        issue_edges(t, t)
    for t in range(2):
        drain_edges(t)
        pltpu.async_copy(h_hbm.at[idxs[t]], rows[t], gsems[t])

    plsc.subcore_barrier()

    def chunk(ci, u):
        drain_rows(gsems[u], u)
        do_scale(rows[u], nrms[u])
        pltpu.async_copy(rows[u], accum_sh.at[dsts[u]], ssems[u], add=True)

        @pl.when(ci + NBUF < NCHUNK)
        def _():
            issue_edges(ci + NBUF, u)

        u2 = (u + 2) % NBUF

        @pl.when((ci + 2 < NCHUNK) & (ci >= 1))
        def _():
            drain_rows(ssems[u2], u2)

        @pl.when(ci + 2 < NCHUNK)
        def _():
            drain_edges(u2)
            pltpu.async_copy(h_hbm.at[idxs[u2]], rows[u2], gsems[u2])

    def pipeline(k, cr):
        for u in range(NBUF):
            chunk(k * NBUF + u, u)
        return cr

    nk = NCHUNK // NBUF
    lax.fori_loop(0, nk, pipeline, 0)
    for ci in range(nk * NBUF, NCHUNK):
        chunk(ci, ci % NBUF)
    for ci in range(NCHUNK - NBUF, NCHUNK):
        drain_rows(ssems[ci % NBUF], ci % NBUF)

    plsc.subcore_barrier()
    dstart = pl.multiple_of(c * N + s * ZSTRIPE, 8)
    pltpu.sync_copy(accum_sh.at[pl.ds(zstart, ZSTRIPE)],
                    part_hbm.at[pl.ds(dstart, ZSTRIPE)])

    @pl.when(s == NS - 1)
    def _():
        tstart = pl.multiple_of(c * N + NS * ZSTRIPE, 8)
        pltpu.sync_copy(accum_sh.at[pl.ds(NS * ZSTRIPE, ZTAIL)],
                        part_hbm.at[pl.ds(tstart, ZTAIL)])


_sc_scatter = pl.kernel(
    _sc_body,
    out_type=jax.ShapeDtypeStruct((NC * N, D), jnp.float32),
    mesh=plsc.VectorSubcoreMesh(core_axis_name="c", subcore_axis_name="s"),
    scratch_types=(
        [pltpu.VMEM((C,), jnp.int32) for _ in range(NBUF)]
        + [pltpu.VMEM((C,), jnp.int32) for _ in range(NBUF)]
        + [pltpu.VMEM((C,), jnp.float32) for _ in range(NBUF)]
        + [pltpu.VMEM((C, D), jnp.float32) for _ in range(NBUF)]
        + [pltpu.VMEM_SHARED((N, D), jnp.float32)]
        + [pltpu.SemaphoreType.DMA for _ in range(3 * NBUF)]
    ),
)


def _combine_body(p0_ref, p1_ref, x_ref, root_ref, out_ref):
    out_ref[...] = (p0_ref[...] + p1_ref[...]
                    + jnp.dot(x_ref[...], root_ref[...],
                              preferred_element_type=jnp.float32))


def _combine(part, x, root):
    bn = 2000
    nb = N // bn
    return pl.pallas_call(
        _combine_body,
        grid=(nb,),
        in_specs=[
            pl.BlockSpec((bn, D), lambda i: (i, 0)),
            pl.BlockSpec((bn, D), lambda i: (nb + i, 0)),
            pl.BlockSpec((bn, D), lambda i: (i, 0)),
            pl.BlockSpec((D, D), lambda i: (0, 0)),
        ],
        out_specs=pl.BlockSpec((bn, D), lambda i: (i, 0)),
        out_shape=jax.ShapeDtypeStruct((N, D), jnp.float32),
    )(part, part, x, root)


def kernel(x, edge_index, edge_type, edge_norm, dim, weight_relation, root):
    h = _compute_h(x, weight_relation)
    h_flat = h.reshape(R * N, D)
    idx = (edge_type.astype(jnp.int32) * N
           + edge_index[1].astype(jnp.int32)).reshape(NW, NCHUNK, C)
    dst = edge_index[0].astype(jnp.int32).reshape(NW, NCHUNK, C)
    nrm = edge_norm.reshape(NW, NCHUNK, C)
    zeros = jnp.zeros((N, D), jnp.float32)
    part = _sc_scatter(h_flat, idx, dst, nrm, zeros)
    return _combine(part, x, root)


# speedup vs baseline: 27.7510x; 1.0045x over previous
import jax
import jax.numpy as jnp
from jax import lax
from jax.experimental import pallas as pl
from jax.experimental.pallas import tpu as pltpu
from jax.experimental.pallas import tpu_sc as plsc

N = 10000
E = 320000
D = 128
R = 8
NC = 2
NS = 16
NW = NC * NS
EW = E // NW
C = 80
NCHUNK = EW // C
NBUF = 3
ZSTRIPE = 624
ZTAIL = N - NS * ZSTRIPE


def _h_body(x_ref, w_ref, h_ref):
    h_ref[0] = jnp.dot(x_ref[...], w_ref[0], preferred_element_type=jnp.float32)


def _compute_h(x_bf, w_bf):
    bn = 2000
    return pl.pallas_call(
        _h_body,
        grid=(R, N // bn),
        in_specs=[
            pl.BlockSpec((bn, D), lambda r, i: (i, 0)),
            pl.BlockSpec((1, D, D), lambda r, i: (r, 0, 0)),
        ],
        out_specs=pl.BlockSpec((1, bn, D), lambda r, i: (r, i, 0)),
        out_shape=jax.ShapeDtypeStruct((R, N, D), jnp.float32),
    )(x_bf, w_bf)


def _sc_body(h_hbm, idx_hbm, dst_hbm, nrm_hbm, zeros_hbm, part_hbm,
             idx0, idx1, idx2, dst0, dst1, dst2, nrm0, nrm1, nrm2,
             rows0, rows1, rows2, accum_sh,
             gsem0, gsem1, gsem2, ssem0, ssem1, ssem2, esem0, esem1, esem2,
             zsem):
    c = lax.axis_index("c")
    s = lax.axis_index("s")
    wid = c * NS + s
    idxs = (idx0, idx1, idx2)
    dsts = (dst0, dst1, dst2)
    nrms = (nrm0, nrm1, nrm2)
    rows = (rows0, rows1, rows2)
    gsems = (gsem0, gsem1, gsem2)
    ssems = (ssem0, ssem1, ssem2)
    esems = (esem0, esem1, esem2)

    zstart = pl.multiple_of(s * ZSTRIPE, 8)
    pltpu.async_copy(zeros_hbm.at[pl.ds(zstart, ZSTRIPE)],
                     accum_sh.at[pl.ds(zstart, ZSTRIPE)], zsem)

    @pl.when(s == NS - 1)
    def _():
        pltpu.async_copy(zeros_hbm.at[pl.ds(NS * ZSTRIPE, ZTAIL)],
                         accum_sh.at[pl.ds(NS * ZSTRIPE, ZTAIL)], zsem)

    def issue_edges(ci, u):
        pltpu.async_copy(idx_hbm.at[wid, ci], idxs[u], esems[u])
        pltpu.async_copy(dst_hbm.at[wid, ci], dsts[u], esems[u])
        pltpu.async_copy(nrm_hbm.at[wid, ci], nrms[u], esems[u])

    def drain_edges(u):
        pltpu.make_async_copy(idx_hbm.at[0, 0], idxs[u], esems[u]).wait()
        pltpu.make_async_copy(dst_hbm.at[0, 0], dsts[u], esems[u]).wait()
        pltpu.make_async_copy(nrm_hbm.at[0, 0], nrms[u], esems[u]).wait()

    def drain_rows(sem, u):
        pltpu.make_async_copy(h_hbm.at[pl.ds(0, C)], rows[u], sem).wait()

    def do_scale(rows_b, nrm_b):
        def scale(g, cr):
            nrm16 = nrm_b[pl.ds(g * 16, 16)]
            for l in range(16):
                e = g * 16 + l
                nv = nrm16[l]
                for j in range(D // 16):
                    sl = pl.ds(j * 16, 16)
                    rows_b[e, sl] = rows_b[e, sl] * nv
            return cr

        lax.fori_loop(0, C // 16, scale, 0)

    for t in range(NBUF):
        issue_edges(t, t)
    for t in range(2):
        drain_edges(t)
        pltpu.async_copy(h_hbm.at[idxs[t]], rows[t], gsems[t])

    pltpu.make_async_copy(zeros_hbm.at[pl.ds(zstart, ZSTRIPE)],
                          accum_sh.at[pl.ds(zstart, ZSTRIPE)], zsem).wait()

    @pl.when(s == NS - 1)
    def _():
        pltpu.make_async_copy(zeros_hbm.at[pl.ds(NS * ZSTRIPE, ZTAIL)],
                              accum_sh.at[pl.ds(NS * ZSTRIPE, ZTAIL)],
                              zsem).wait()

    plsc.subcore_barrier()

    def chunk(ci, u):
        drain_rows(gsems[u], u)
        do_scale(rows[u], nrms[u])
        pltpu.async_copy(rows[u], accum_sh.at[dsts[u]], ssems[u], add=True)

        @pl.when(ci + NBUF < NCHUNK)
        def _():
            issue_edges(ci + NBUF, u)

        u2 = (u + 2) % NBUF

        @pl.when((ci + 2 < NCHUNK) & (ci >= 1))
        def _():
            drain_rows(ssems[u2], u2)

        @pl.when(ci + 2 < NCHUNK)
        def _():
            drain_edges(u2)
            pltpu.async_copy(h_hbm.at[idxs[u2]], rows[u2], gsems[u2])

    def pipeline(k, cr):
        for u in range(NBUF):
            chunk(k * NBUF + u, u)
        return cr

    nk = NCHUNK // NBUF
    lax.fori_loop(0, nk, pipeline, 0)
    for ci in range(nk * NBUF, NCHUNK):
        chunk(ci, ci % NBUF)
    for ci in range(NCHUNK - NBUF, NCHUNK):
        drain_rows(ssems[ci % NBUF], ci % NBUF)

    plsc.subcore_barrier()
    dstart = pl.multiple_of(c * N + s * ZSTRIPE, 8)
    pltpu.sync_copy(accum_sh.at[pl.ds(zstart, ZSTRIPE)],
                    part_hbm.at[pl.ds(dstart, ZSTRIPE)])

    @pl.when(s == NS - 1)
    def _():
        tstart = pl.multiple_of(c * N + NS * ZSTRIPE, 8)
        pltpu.sync_copy(accum_sh.at[pl.ds(NS * ZSTRIPE, ZTAIL)],
                        part_hbm.at[pl.ds(tstart, ZTAIL)])


_sc_scatter = pl.kernel(
    _sc_body,
    out_type=jax.ShapeDtypeStruct((NC * N, D), jnp.float32),
    mesh=plsc.VectorSubcoreMesh(core_axis_name="c", subcore_axis_name="s"),
    scratch_types=(
        [pltpu.VMEM((C,), jnp.int32) for _ in range(NBUF)]
        + [pltpu.VMEM((C,), jnp.int32) for _ in range(NBUF)]
        + [pltpu.VMEM((C,), jnp.float32) for _ in range(NBUF)]
        + [pltpu.VMEM((C, D), jnp.float32) for _ in range(NBUF)]
        + [pltpu.VMEM_SHARED((N, D), jnp.float32)]
        + [pltpu.SemaphoreType.DMA for _ in range(3 * NBUF + 1)]
    ),
)


def _combine_body(p0_ref, p1_ref, x_ref, root_ref, out_ref):
    out_ref[...] = (p0_ref[...] + p1_ref[...]
                    + jnp.dot(x_ref[...], root_ref[...],
                              preferred_element_type=jnp.float32))


def _combine(part, x, root):
    bn = 2000
    nb = N // bn
    return pl.pallas_call(
        _combine_body,
        grid=(nb,),
        in_specs=[
            pl.BlockSpec((bn, D), lambda i: (i, 0)),
            pl.BlockSpec((bn, D), lambda i: (nb + i, 0)),
            pl.BlockSpec((bn, D), lambda i: (i, 0)),
            pl.BlockSpec((D, D), lambda i: (0, 0)),
        ],
        out_specs=pl.BlockSpec((bn, D), lambda i: (i, 0)),
        out_shape=jax.ShapeDtypeStruct((N, D), jnp.float32),
    )(part, part, x, root)


def kernel(x, edge_index, edge_type, edge_norm, dim, weight_relation, root):
    h = _compute_h(x.astype(jnp.bfloat16), weight_relation.astype(jnp.bfloat16))
    h_flat = h.reshape(R * N, D)
    idx = (edge_type.astype(jnp.int32) * N
           + edge_index[1].astype(jnp.int32)).reshape(NW, NCHUNK, C)
    dst = edge_index[0].astype(jnp.int32).reshape(NW, NCHUNK, C)
    nrm = edge_norm.reshape(NW, NCHUNK, C)
    zeros = jnp.zeros((N, D), jnp.float32)
    part = _sc_scatter(h_flat, idx, dst, nrm, zeros)
    return _combine(part, x, root)


# speedup vs baseline: 29.9371x; 1.0788x over previous
import jax
import jax.numpy as jnp
from jax import lax
from jax.experimental import pallas as pl
from jax.experimental.pallas import tpu as pltpu
from jax.experimental.pallas import tpu_sc as plsc

N = 10000
E = 320000
D = 128
R = 8
NC = 2
NS = 16
NW = NC * NS
EW = E // NW
C = 80
NCHUNK = EW // C
NBUF = 3
ZSTRIPE = 624
ZTAIL = N - NS * ZSTRIPE


def _h_body(x_ref, w_ref, p_ref):
    lo = jnp.dot(x_ref[...], w_ref[0], preferred_element_type=jnp.float32)
    hi = jnp.dot(x_ref[...], w_ref[1], preferred_element_type=jnp.float32)
    lo16 = jax.lax.bitcast_convert_type(
        lo.astype(jnp.bfloat16), jnp.uint16).astype(jnp.uint32)
    hi16 = jax.lax.bitcast_convert_type(
        hi.astype(jnp.bfloat16), jnp.uint16).astype(jnp.uint32)
    p_ref[0] = jax.lax.bitcast_convert_type(lo16 | (hi16 << 16), jnp.float32)


def _compute_packed(x_bf, w_bf):
    bn = 2000
    return pl.pallas_call(
        _h_body,
        grid=(R // 2, N // bn),
        in_specs=[
            pl.BlockSpec((bn, D), lambda p, i: (i, 0)),
            pl.BlockSpec((2, D, D), lambda p, i: (p, 0, 0)),
        ],
        out_specs=pl.BlockSpec((1, bn, D), lambda p, i: (p, i, 0)),
        out_shape=jax.ShapeDtypeStruct((R // 2, N, D), jnp.float32),
    )(x_bf, w_bf)


def _sc_body(h_hbm, et_hbm, src_hbm, dst_hbm, nrm_hbm, zeros_hbm, part_hbm,
             et0, et1, et2, src0, src1, src2, dst0, dst1, dst2,
             nrm0, nrm1, nrm2, idxq0, idxq1, idxq2,
             rows0, rows1, rows2, accum_sh,
             gsem0, gsem1, gsem2, ssem0, ssem1, ssem2, esem0, esem1, esem2,
             zsem):
    c = lax.axis_index("c")
    s = lax.axis_index("s")
    wid = c * NS + s
    ets = (et0, et1, et2)
    srcs = (src0, src1, src2)
    dsts = (dst0, dst1, dst2)
    nrms = (nrm0, nrm1, nrm2)
    idxqs = (idxq0, idxq1, idxq2)
    rows = (rows0, rows1, rows2)
    gsems = (gsem0, gsem1, gsem2)
    ssems = (ssem0, ssem1, ssem2)
    esems = (esem0, esem1, esem2)

    zstart = pl.multiple_of(s * ZSTRIPE, 8)
    pltpu.async_copy(zeros_hbm.at[pl.ds(zstart, ZSTRIPE)],
                     accum_sh.at[pl.ds(zstart, ZSTRIPE)], zsem)

    @pl.when(s == NS - 1)
    def _():
        pltpu.async_copy(zeros_hbm.at[pl.ds(NS * ZSTRIPE, ZTAIL)],
                         accum_sh.at[pl.ds(NS * ZSTRIPE, ZTAIL)], zsem)

    base_w = wid * EW

    def issue_edges(ci, u):
        base = base_w + ci * C
        pltpu.async_copy(et_hbm.at[pl.ds(base, C)], ets[u], esems[u])
        pltpu.async_copy(src_hbm.at[pl.ds(base, C)], srcs[u], esems[u])
        pltpu.async_copy(dst_hbm.at[pl.ds(base, C)], dsts[u], esems[u])
        pltpu.async_copy(nrm_hbm.at[pl.ds(base, C)], nrms[u], esems[u])

    def drain_edges(u):
        pltpu.make_async_copy(et_hbm.at[pl.ds(0, C)], ets[u], esems[u]).wait()
        pltpu.make_async_copy(et_hbm.at[pl.ds(0, C)], srcs[u], esems[u]).wait()
        pltpu.make_async_copy(et_hbm.at[pl.ds(0, C)], dsts[u], esems[u]).wait()
        pltpu.make_async_copy(nrm_hbm.at[pl.ds(0, C)], nrms[u], esems[u]).wait()

    def compute_idx(u):
        for g in range(C // 16):
            sl = pl.ds(g * 16, 16)
            idxqs[u][sl] = (ets[u][sl] >> 1) * N + srcs[u][sl]

    def drain_rows(sem, u):
        pltpu.make_async_copy(h_hbm.at[pl.ds(0, C)], rows[u], sem).wait()

    def do_scale(rows_b, nrm_b, et_b):
        def scale(g, cr):
            sl16 = pl.ds(g * 16, 16)
            nrm16 = nrm_b[sl16]
            par16 = (et_b[sl16] << 31)
            for l in range(16):
                e = g * 16 + l
                nv = nrm16[l]
                hi_sel = par16[l] < 0
                for q in range(D // 16):
                    sl = pl.ds(q * 16, 16)
                    w = jax.lax.bitcast_convert_type(rows_b[e, sl], jnp.int32)
                    lo = jax.lax.bitcast_convert_type(w << 16, jnp.float32)
                    hi = jax.lax.bitcast_convert_type(w & jnp.int32(-65536),
                                                      jnp.float32)
                    val = jnp.where(hi_sel, hi, lo)
                    rows_b[e, sl] = val * nv
            return cr

        lax.fori_loop(0, C // 16, scale, 0)

    for t in range(NBUF):
        issue_edges(t, t)
    for t in range(2):
        drain_edges(t)
        compute_idx(t)
        pltpu.async_copy(h_hbm.at[idxqs[t]], rows[t], gsems[t])

    pltpu.make_async_copy(zeros_hbm.at[pl.ds(zstart, ZSTRIPE)],
                          accum_sh.at[pl.ds(zstart, ZSTRIPE)], zsem).wait()

    @pl.when(s == NS - 1)
    def _():
        pltpu.make_async_copy(zeros_hbm.at[pl.ds(NS * ZSTRIPE, ZTAIL)],
                              accum_sh.at[pl.ds(NS * ZSTRIPE, ZTAIL)],
                              zsem).wait()

    plsc.subcore_barrier()

    def chunk(ci, u):
        drain_rows(gsems[u], u)
        do_scale(rows[u], nrms[u], ets[u])
        pltpu.async_copy(rows[u], accum_sh.at[dsts[u]], ssems[u], add=True)

        @pl.when(ci + NBUF < NCHUNK)
        def _():
            issue_edges(ci + NBUF, u)

        u2 = (u + 2) % NBUF

        @pl.when((ci + 2 < NCHUNK) & (ci >= 1))
        def _():
            drain_rows(ssems[u2], u2)

        @pl.when(ci + 2 < NCHUNK)
        def _():
            drain_edges(u2)
            compute_idx(u2)
            pltpu.async_copy(h_hbm.at[idxqs[u2]], rows[u2], gsems[u2])

    def pipeline(k, cr):
        for u in range(NBUF):
            chunk(k * NBUF + u, u)
        return cr

    nk = NCHUNK // NBUF
    lax.fori_loop(0, nk, pipeline, 0)
    for ci in range(nk * NBUF, NCHUNK):
        chunk(ci, ci % NBUF)
    for ci in range(NCHUNK - NBUF, NCHUNK):
        drain_rows(ssems[ci % NBUF], ci % NBUF)

    plsc.subcore_barrier()
    dstart = pl.multiple_of(c * N + s * ZSTRIPE, 8)
    pltpu.sync_copy(accum_sh.at[pl.ds(zstart, ZSTRIPE)],
                    part_hbm.at[pl.ds(dstart, ZSTRIPE)])

    @pl.when(s == NS - 1)
    def _():
        tstart = pl.multiple_of(c * N + NS * ZSTRIPE, 8)
        pltpu.sync_copy(accum_sh.at[pl.ds(NS * ZSTRIPE, ZTAIL)],
                        part_hbm.at[pl.ds(tstart, ZTAIL)])


_sc_scatter = pl.kernel(
    _sc_body,
    out_type=jax.ShapeDtypeStruct((NC * N, D), jnp.float32),
    mesh=plsc.VectorSubcoreMesh(core_axis_name="c", subcore_axis_name="s"),
    scratch_types=(
        [pltpu.VMEM((C,), jnp.int32) for _ in range(3 * NBUF)]
        + [pltpu.VMEM((C,), jnp.float32) for _ in range(NBUF)]
        + [pltpu.VMEM((C,), jnp.int32) for _ in range(NBUF)]
        + [pltpu.VMEM((C, D), jnp.float32) for _ in range(NBUF)]
        + [pltpu.VMEM_SHARED((N, D), jnp.float32)]
        + [pltpu.SemaphoreType.DMA for _ in range(3 * NBUF + 1)]
    ),
)


def _combine_body(p0_ref, p1_ref, x_ref, root_ref, out_ref):
    out_ref[...] = (p0_ref[...] + p1_ref[...]
                    + jnp.dot(x_ref[...], root_ref[...],
                              preferred_element_type=jnp.float32))


def _combine(part, x, root):
    bn = 2000
    nb = N // bn
    return pl.pallas_call(
        _combine_body,
        grid=(nb,),
        in_specs=[
            pl.BlockSpec((bn, D), lambda i: (i, 0)),
            pl.BlockSpec((bn, D), lambda i: (nb + i, 0)),
            pl.BlockSpec((bn, D), lambda i: (i, 0)),
            pl.BlockSpec((D, D), lambda i: (0, 0)),
        ],
        out_specs=pl.BlockSpec((bn, D), lambda i: (i, 0)),
        out_shape=jax.ShapeDtypeStruct((N, D), jnp.float32),
    )(part, part, x, root)


def kernel(x, edge_index, edge_type, edge_norm, dim, weight_relation, root):
    pk = _compute_packed(x.astype(jnp.bfloat16),
                         weight_relation.astype(jnp.bfloat16))
    pk_flat = pk.reshape(R // 2 * N, D)
    zeros = jnp.zeros((N, D), jnp.float32)
    part = _sc_scatter(pk_flat, edge_type.astype(jnp.int32),
                       edge_index[1].astype(jnp.int32),
                       edge_index[0].astype(jnp.int32), edge_norm, zeros)
    return _combine(part, x, root)
